# baseline (speedup 1.0000x reference)
import functools

import jax
import jax.numpy as jnp
from jax import lax
from jax.experimental import pallas as pl
from jax.experimental.pallas import tpu as pltpu
from jax.experimental.pallas import tpu_sc as plsc


def _round_up(x, m):
    return (x + m - 1) // m * m


def kernel(atoms, bond_graph, atom_kind_ohe, atom_mass_lookup):
    N, K = bond_graph.shape
    W = atom_kind_ohe.shape[1]

    info = plsc.get_sparse_core_info()
    NC, NS, L = info.num_cores, info.num_subcores, info.num_lanes
    NW = NC * NS

    assert N % L == 0
    n_chunks = N // L
    J = -(-n_chunks // NW)
    C = J * L

    atoms_flat = atoms.reshape(-1).astype(jnp.int32)
    bg = bond_graph.astype(jnp.int32)
    ohe_flat = atom_kind_ohe.reshape(-1).astype(jnp.float32)
    ohe_p = jnp.zeros((_round_up(ohe_flat.shape[0], L),), jnp.float32).at[
        : ohe_flat.shape[0]
    ].set(ohe_flat)
    mass_p = jnp.zeros((_round_up(atom_mass_lookup.shape[0], L),), jnp.int32).at[
        : atom_mass_lookup.shape[0]
    ].set(atom_mass_lookup.astype(jnp.int32))

    mesh = plsc.VectorSubcoreMesh(core_axis_name="c", subcore_axis_name="s")

    @functools.partial(
        pl.kernel,
        mesh=mesh,
        out_type=jax.ShapeDtypeStruct((N, 7), jnp.float32),
        compiler_params=pltpu.CompilerParams(needs_layout_passes=False),
        scratch_types=[
            pltpu.VMEM((C,), jnp.int32),
            pltpu.VMEM((C * K,), jnp.int32),
            pltpu.VMEM((C * K,), jnp.int32),
            pltpu.VMEM((C, 7), jnp.float32),
            pltpu.VMEM(ohe_p.shape, jnp.float32),
            pltpu.VMEM(mass_p.shape, jnp.int32),
            pltpu.SemaphoreType.DMA,
            pltpu.SemaphoreType.DMA,
        ],
    )
    def run(atoms_hbm, bg_hbm, ohe_hbm, mass_hbm, out_hbm,
            own_v, bg_v, na_v, out_v, ohe_v, mass_v, sem, sem2):
        wid = lax.axis_index("s") * NC + lax.axis_index("c")
        base = jnp.minimum(wid * C, N - C)

        c1 = pltpu.make_async_copy(atoms_hbm.at[pl.ds(base, C)], own_v, sem)
        c2 = pltpu.make_async_copy(bg_hbm.at[pl.ds(base * K, C * K)], bg_v, sem)
        c3 = pltpu.make_async_copy(ohe_hbm, ohe_v, sem)
        c4 = pltpu.make_async_copy(mass_hbm, mass_v, sem)
        c1.start(); c2.start(); c3.start(); c4.start()
        c2.wait()
        GW = 128
        cgs = [
            pltpu.make_async_copy(
                atoms_hbm.at[bg_v.at[pl.ds(i * GW, GW)]],
                na_v.at[pl.ds(i * GW, GW)], sem2)
            for i in range(C * K // GW)
        ]
        for cg in cgs:
            cg.start()
        c1.wait(); c3.wait(); c4.wait()

        lane = lax.iota(jnp.int32, L)
        deg = jnp.full((L,), float(K) / 4.0, jnp.float32)

        @plsc.parallel_loop(0, C, L, unroll=1)
        def _(off):
            r = off + lane
            a = own_v[pl.ds(off, L)]
            aw = a * W
            for c in range(W):
                col = plsc.load_gather(ohe_v, [aw + c])
                plsc.store_scatter(out_v, [r, jnp.full((L,), c, jnp.int32)], col)
            m = plsc.load_gather(mass_v, [a]).astype(jnp.float32) * (1.0 / 16.0)
            plsc.store_scatter(out_v, [r, jnp.full((L,), W, jnp.int32)], m)
            plsc.store_scatter(out_v, [r, jnp.full((L,), W + 1, jnp.int32)], deg)

        for cg in cgs:
            cg.wait()

        @plsc.parallel_loop(0, C, L, unroll=1)
        def _(off):
            r = off + lane
            q0 = (off + lane) * K
            na = []
            for k in range(K):
                v = plsc.load_gather(na_v, [q0 + k])
                na.append((v == 1).astype(jnp.int32))
            h = (na[0] + na[1]) + (na[2] + na[3]) if K == 4 else sum(na)
            plsc.store_scatter(out_v, [r, jnp.full((L,), W + 2, jnp.int32)],
                               h.astype(jnp.float32) * 0.25)

        pltpu.sync_copy(out_v, out_hbm.at[pl.ds(base, C)])

    return run(atoms_flat, bg.reshape(-1), ohe_p, mass_p)


# speedup vs baseline: 1.0897x; 1.0897x over previous
import functools

import jax
import jax.numpy as jnp
from jax import lax
from jax.experimental import pallas as pl
from jax.experimental.pallas import tpu as pltpu
from jax.experimental.pallas import tpu_sc as plsc


def _round_up(x, m):
    return (x + m - 1) // m * m


def kernel(atoms, bond_graph, atom_kind_ohe, atom_mass_lookup):
    N, K = bond_graph.shape
    W = atom_kind_ohe.shape[1]

    info = plsc.get_sparse_core_info()
    NC, NS, L = info.num_cores, info.num_subcores, info.num_lanes
    NW = NC * NS

    assert N % L == 0
    n_chunks = N // L
    J = -(-n_chunks // NW)
    C = J * L

    atoms_flat = atoms.reshape(-1).astype(jnp.int32)
    bg = bond_graph.astype(jnp.int32)
    ohe_flat = atom_kind_ohe.reshape(-1).astype(jnp.float32)
    ohe_p = jnp.zeros((_round_up(ohe_flat.shape[0], L),), jnp.float32).at[
        : ohe_flat.shape[0]
    ].set(ohe_flat)
    mass_p = jnp.zeros((_round_up(atom_mass_lookup.shape[0], L),), jnp.int32).at[
        : atom_mass_lookup.shape[0]
    ].set(atom_mass_lookup.astype(jnp.int32))

    mesh = plsc.VectorSubcoreMesh(core_axis_name="c", subcore_axis_name="s")

    @functools.partial(
        pl.kernel,
        mesh=mesh,
        out_type=jax.ShapeDtypeStruct((N, 7), jnp.float32),
        compiler_params=pltpu.CompilerParams(needs_layout_passes=False),
        scratch_types=[
            pltpu.VMEM((N,), jnp.int32),
            pltpu.VMEM((C, K), jnp.int32),
            pltpu.VMEM((C, 7), jnp.float32),
            pltpu.VMEM(ohe_p.shape, jnp.float32),
            pltpu.VMEM(mass_p.shape, jnp.int32),
            pltpu.SemaphoreType.DMA,
        ],
    )
    def run(atoms_hbm, bg_hbm, ohe_hbm, mass_hbm, out_hbm,
            atoms_v, bg_v, out_v, ohe_v, mass_v, sem):
        wid = lax.axis_index("s") * NC + lax.axis_index("c")
        base = jnp.minimum(wid * C, N - C)

        c1 = pltpu.make_async_copy(atoms_hbm, atoms_v, sem)
        c2 = pltpu.make_async_copy(bg_hbm.at[pl.ds(base, C)], bg_v, sem)
        c3 = pltpu.make_async_copy(ohe_hbm, ohe_v, sem)
        c4 = pltpu.make_async_copy(mass_hbm, mass_v, sem)
        c1.start(); c2.start(); c3.start(); c4.start()
        c1.wait(); c2.wait(); c3.wait(); c4.wait()

        lane = lax.iota(jnp.int32, L)
        deg = jnp.full((L,), float(K) / 4.0, jnp.float32)

        @plsc.parallel_loop(0, C, L, unroll=1)
        def _(off):
            r = off + lane
            a = atoms_v[pl.ds(base + off, L)]
            aw = a * W
            for c in range(W):
                col = plsc.load_gather(ohe_v, [aw + c])
                plsc.store_scatter(out_v, [r, jnp.full((L,), c, jnp.int32)], col)
            m = plsc.load_gather(mass_v, [a]).astype(jnp.float32) * (1.0 / 16.0)
            plsc.store_scatter(out_v, [r, jnp.full((L,), W, jnp.int32)], m)
            plsc.store_scatter(out_v, [r, jnp.full((L,), W + 1, jnp.int32)], deg)
            na = []
            for k in range(K):
                nb = plsc.load_gather(bg_v, [r, jnp.full((L,), k, jnp.int32)])
                na.append((plsc.load_gather(atoms_v, [nb]) == 1).astype(jnp.int32))
            h = (na[0] + na[1]) + (na[2] + na[3]) if K == 4 else sum(na)
            plsc.store_scatter(out_v, [r, jnp.full((L,), W + 2, jnp.int32)],
                               h.astype(jnp.float32) * 0.25)

        pltpu.sync_copy(out_v, out_hbm.at[pl.ds(base, C)])

    return run(atoms_flat, bg, ohe_p, mass_p)
